# baseline (speedup 1.0000x reference)
import functools

import jax
import jax.numpy as jnp
from jax.experimental import pallas as pl
from jax.experimental.pallas import tpu as pltpu

_N = 10000
_E = 320000
_G = 64


def _elu(x):
    return jnp.where(x > 0, x, jnp.exp(jnp.minimum(x, 0.0)) - 1.0)


def _mlp3_body(x_ref, w1, b1, w2, b2, w3, b3, o_ref, *, elu_out):
    h = x_ref[...]
    h = _elu(jnp.dot(h, w1[...], preferred_element_type=jnp.float32) + b1[...])
    h = _elu(jnp.dot(h, w2[...], preferred_element_type=jnp.float32) + b2[...])
    h = jnp.dot(h, w3[...], preferred_element_type=jnp.float32) + b3[...]
    if elu_out:
        h = _elu(h)
    o_ref[...] = h


def _mlp3(x, params, elu_out=False, block=2048):
    (w1, b1), (w2, b2), (w3, b3) = params
    m, k = x.shape
    out_dim = w3.shape[1]
    mp = ((m + block - 1) // block) * block
    if mp != m:
        x = jnp.pad(x, ((0, mp - m), (0, 0)))
    grid = mp // block
    full = lambda r, c: pl.BlockSpec((r, c), lambda i: (0, 0))
    out = pl.pallas_call(
        functools.partial(_mlp3_body, elu_out=elu_out),
        grid=(grid,),
        in_specs=[
            pl.BlockSpec((block, k), lambda i: (i, 0)),
            full(*w1.shape), full(1, b1.shape[0]),
            full(*w2.shape), full(1, b2.shape[0]),
            full(*w3.shape), full(1, b3.shape[0]),
        ],
        out_specs=pl.BlockSpec((block, out_dim), lambda i: (i, 0)),
        out_shape=jax.ShapeDtypeStruct((mp, out_dim), jnp.float32),
    )(x, w1, b1.reshape(1, -1), w2, b2.reshape(1, -1), w3, b3.reshape(1, -1))
    return out[:m]


def _conv_layer(feat, pos, src, dst, local_p, global_p):
    msg_in = jnp.concatenate([feat[src], pos[src] - pos[dst]], axis=1)
    msg = _mlp3(msg_in, local_p)
    agg = jax.ops.segment_max(msg, dst, num_segments=_N)
    return _mlp3(agg, global_p, elu_out=True)


def kernel(x, pos, params, edge_index, batch):
    loop = jnp.arange(_N, dtype=edge_index.dtype)
    src = jnp.concatenate([edge_index[0], loop])
    dst = jnp.concatenate([edge_index[1], loop])

    x1 = _conv_layer(x, pos, src, dst, params['ln1'], params['gn1'])
    x2 = _conv_layer(x1, pos, src, dst, params['ln2'], params['gn2'])

    x_add = jax.ops.segment_sum(x2, batch, num_segments=_G)
    cnt = jax.ops.segment_sum(jnp.ones((_N, 1), jnp.float32), batch, num_segments=_G)
    x_mean = x_add / jnp.maximum(cnt, 1.0)
    x_max = jax.ops.segment_max(x2, batch, num_segments=_G)
    h = jnp.concatenate([x_max, x_mean, x_add], axis=1)
    wl, bl = params['lin1']
    return h @ wl + bl


# speedup vs baseline: 1.3629x; 1.3629x over previous
import functools

import jax
import jax.numpy as jnp
from jax import lax
from jax.experimental import pallas as pl
from jax.experimental.pallas import tpu as pltpu
from jax.experimental.pallas import tpu_sc as plsc

_N = 10000
_E = 320000
_G = 64
_NW = 32
_RPW = 320
_NPAD = _NW * _RPW
_SENTINEL = 1 << 29


def _elu(x):
    return jnp.where(x > 0, x, jnp.exp(jnp.minimum(x, 0.0)) - 1.0)


def _mlp3_body(x_ref, w1, b1, w2, b2, w3, b3, o_ref, *, elu_out):
    h = x_ref[...]
    h = _elu(jnp.dot(h, w1[...], preferred_element_type=jnp.float32) + b1[...])
    h = _elu(jnp.dot(h, w2[...], preferred_element_type=jnp.float32) + b2[...])
    h = jnp.dot(h, w3[...], preferred_element_type=jnp.float32) + b3[...]
    if elu_out:
        h = _elu(h)
    o_ref[...] = h


def _mlp3(x, params, elu_out=False, block=2048, pad_to=None, trim=True):
    (w1, b1), (w2, b2), (w3, b3) = params
    m, k = x.shape
    out_dim = w3.shape[1]
    mp = pad_to or ((m + block - 1) // block) * block
    assert mp % block == 0
    if mp != m:
        x = jnp.pad(x, ((0, mp - m), (0, 0)))
    grid = mp // block
    full = lambda r, c: pl.BlockSpec((r, c), lambda i: (0, 0))
    out = pl.pallas_call(
        functools.partial(_mlp3_body, elu_out=elu_out),
        grid=(grid,),
        in_specs=[
            pl.BlockSpec((block, k), lambda i: (i, 0)),
            full(*w1.shape), full(1, b1.shape[0]),
            full(*w2.shape), full(1, b2.shape[0]),
            full(*w3.shape), full(1, b3.shape[0]),
        ],
        out_specs=pl.BlockSpec((block, out_dim), lambda i: (i, 0)),
        out_shape=jax.ShapeDtypeStruct((mp, out_dim), jnp.float32),
    )(x, w1, b1.reshape(1, -1), w2, b2.reshape(1, -1), w3, b3.reshape(1, -1))
    return out[:m] if trim else out


def _segmax_sc(msg, dst):
    mp, d = msg.shape
    ch = 4096
    fb = 128
    assert mp % ch == 0 and d % 16 == 0
    n_chunks = mp // ch
    ncol = d // 16
    mesh = plsc.VectorSubcoreMesh(core_axis_name="c", subcore_axis_name="s")

    @functools.partial(
        pl.kernel,
        mesh=mesh,
        compiler_params=pltpu.CompilerParams(needs_layout_passes=False),
        out_type=jax.ShapeDtypeStruct((_NPAD, d), jnp.float32),
        scratch_types=[
            pltpu.VMEM((_RPW + 1, d), jnp.float32),
            pltpu.VMEM((ch,), jnp.int32),
            pltpu.VMEM((fb,), jnp.int32),
            pltpu.VMEM((fb,), jnp.int32),
            pltpu.VMEM((fb, d), jnp.float32),
            pltpu.SemaphoreType.DMA,
        ],
    )
    def k(msg_hbm, dst_hbm, out_hbm, acc, dstv, fid, rid, rows, sem):
        wid = lax.axis_index("c") * 16 + lax.axis_index("s")
        base = wid * _RPW
        lanes = lax.iota(jnp.int32, 16)
        neg = jnp.full((16,), -jnp.inf, jnp.float32)

        def init_row(r, _):
            for kk in range(ncol):
                acc[r, pl.ds(kk * 16, 16)] = neg
            return 0
        lax.fori_loop(0, _RPW + 1, init_row, 0)

        def reset_bufs():
            for t in range(fb // 16):
                fid[pl.ds(t * 16, 16)] = t * 16 + lanes
                rid[pl.ds(t * 16, 16)] = jnp.full((16,), _RPW, jnp.int32)
        reset_bufs()

        def flush(_):
            pltpu.async_copy(msg_hbm.at[fid], rows, sem).wait()

            def acc_row(j16, _):
                rv = rid[pl.ds(j16 * 16, 16)]
                for l in range(16):
                    r = rv[l]
                    j = j16 * 16 + l
                    for kk in range(ncol):
                        sl = pl.ds(kk * 16, 16)
                        acc[r, sl] = jnp.maximum(acc[r, sl], rows[j, sl])
                return 0
            lax.fori_loop(0, fb // 16, acc_row, 0)
            reset_bufs()
            return jnp.int32(0)

        def chunk(c, cnt):
            pltpu.sync_copy(dst_hbm.at[pl.ds(c * ch, ch)], dstv)

            def group_c(g, cnt):
                cnt = lax.cond(cnt > fb - 16, flush, lambda x: x, cnt)
                v = dstv[pl.ds(g * 16, 16)]
                rel = v - base
                m = (rel >= 0) & (rel < _RPW)
                eid = c * ch + g * 16 + lanes
                cum = jnp.where(m, 1, 0)
                for sh in (1, 2, 4, 8):
                    prev = jnp.take(cum, jnp.maximum(lanes - sh, 0))
                    cum = cum + jnp.where(lanes >= sh, prev, 0)
                pos = cnt + cum - 1
                plsc.store_scatter(fid, [pos], eid, mask=m)
                plsc.store_scatter(rid, [pos], rel, mask=m)
                return cnt + cum[15]

            return lax.fori_loop(0, ch // 16, group_c, cnt)

        cnt = lax.fori_loop(0, n_chunks, chunk, jnp.int32(0))
        flush(cnt)
        pltpu.sync_copy(acc.at[pl.ds(0, _RPW)], out_hbm.at[pl.ds(base, _RPW)])

    return k(msg, dst)


_MP = 331776


def _conv_layer(feat, pos, src, dst_pad, local_p, global_p):
    msg_in = jnp.concatenate([feat[src], pos[src] - pos[dst_pad[: src.shape[0]]]], axis=1)
    msg = _mlp3(msg_in, local_p, pad_to=_MP, trim=False)
    agg = _segmax_sc(msg, dst_pad)[:_N]
    return _mlp3(agg, global_p, elu_out=True)


def kernel(x, pos, params, edge_index, batch):
    loop = jnp.arange(_N, dtype=edge_index.dtype)
    src = jnp.concatenate([edge_index[0], loop])
    dst_pad = jnp.concatenate([
        edge_index[1], loop,
        jnp.full((_MP - _E - _N,), _SENTINEL, edge_index.dtype),
    ])

    x1 = _conv_layer(x, pos, src, dst_pad, params['ln1'], params['gn1'])
    x2 = _conv_layer(x1, pos, src, dst_pad, params['ln2'], params['gn2'])

    x_add = jax.ops.segment_sum(x2, batch, num_segments=_G)
    cnt = jax.ops.segment_sum(jnp.ones((_N, 1), jnp.float32), batch, num_segments=_G)
    x_mean = x_add / jnp.maximum(cnt, 1.0)
    x_max = jax.ops.segment_max(x2, batch, num_segments=_G)
    h = jnp.concatenate([x_max, x_mean, x_add], axis=1)
    wl, bl = params['lin1']
    return h @ wl + bl
